# baseline (speedup 1.0000x reference)
import functools

import jax
import jax.numpy as jnp
from jax import lax
from jax.experimental import pallas as pl
from jax.experimental.pallas import tpu as pltpu
from jax.experimental.pallas import tpu_sc as plsc

HIDDEN = 64
OUT_DIM = 64
EPS = 1e-12

_NC = 2
_NS = 16
_NW = _NC * _NS

_CHUNK = 128


def _make_sc_gather(vocab, n_rows, d):
    assert n_rows % (_NW * _CHUNK) == 0
    per_w = n_rows // _NW
    n_chunks = per_w // _CHUNK
    mesh = plsc.VectorSubcoreMesh(core_axis_name="c", subcore_axis_name="s")

    @functools.partial(
        pl.kernel,
        mesh=mesh,
        out_type=jax.ShapeDtypeStruct((n_rows, d), jnp.float32),
        scratch_types=[
            pltpu.VMEM((_CHUNK,), jnp.int32),
            pltpu.VMEM((_CHUNK, d), jnp.float32),
            pltpu.SemaphoreType.DMA,
        ],
        compiler_params=pltpu.CompilerParams(use_tc_tiling_on_sc=False),
    )
    def sc_gather(table_hbm, idx_hbm, out_hbm, idx_v, rows_v, sem):
        wid = lax.axis_index("s") * _NC + lax.axis_index("c")
        base = wid * per_w

        def body(j, carry):
            off = base + j * _CHUNK
            pltpu.sync_copy(idx_hbm.at[pl.ds(off, _CHUNK)], idx_v)
            pltpu.async_copy(table_hbm.at[idx_v], rows_v, sem).wait()
            pltpu.sync_copy(rows_v, out_hbm.at[pl.ds(off, _CHUNK)])
            return carry

        lax.fori_loop(0, n_chunks, body, 0)

    return sc_gather


def _ln_proj_body(e_ref, lnw_ref, lnb_ref, wt_ref, b_ref, o_ref):
    e = e_ref[...]
    u = jnp.mean(e, axis=1, keepdims=True)
    d = e - u
    s = jnp.mean(d * d, axis=1, keepdims=True)
    x = d * lax.rsqrt(s + EPS)
    x = x * lnw_ref[...] + lnb_ref[...]
    o_ref[...] = (
        jnp.dot(x, wt_ref[...], preferred_element_type=jnp.float32) + b_ref[...]
    )


def _ln_proj(e, ln_weight, ln_bias, fc_w, fc_b, block_rows=2048):
    n, h = e.shape
    assert n % block_rows == 0
    grid = (n // block_rows,)
    return pl.pallas_call(
        _ln_proj_body,
        grid=grid,
        in_specs=[
            pl.BlockSpec((block_rows, h), lambda i: (i, 0)),
            pl.BlockSpec((1, h), lambda i: (0, 0)),
            pl.BlockSpec((1, h), lambda i: (0, 0)),
            pl.BlockSpec((h, OUT_DIM), lambda i: (0, 0)),
            pl.BlockSpec((1, OUT_DIM), lambda i: (0, 0)),
        ],
        out_specs=pl.BlockSpec((block_rows, OUT_DIM), lambda i: (i, 0)),
        out_shape=jax.ShapeDtypeStruct((n, OUT_DIM), jnp.float32),
    )(e, ln_weight.reshape(1, h), ln_bias.reshape(1, h),
      fc_w.T, fc_b.reshape(1, OUT_DIM))


def kernel(flat_input_ids, tag_table, ln_weight, ln_bias, fc_w, fc_b):
    b, l = flat_input_ids.shape
    vocab, h = tag_table.shape
    idx = flat_input_ids.reshape(-1).astype(jnp.int32)
    gather = _make_sc_gather(vocab, b * l, h)
    e = gather(tag_table, idx)
    out = _ln_proj(e, ln_weight, ln_bias, fc_w, fc_b)
    return out.reshape(b, l, OUT_DIM)
